# baseline (speedup 1.0000x reference)
import jax
import jax.numpy as jnp
from jax.experimental import pallas as pl
from jax.experimental.pallas import tpu as pltpu
from jax.experimental.pallas import tpu_sc as plsc

_N_CLASSES = 1000
_LANES = 16
_R = 32


def kernel(x):
    b, s = x.shape
    rows = b * s
    n_workers = 32
    rows_per_worker = rows // n_workers
    n_chunks = rows_per_worker // _R
    chunk_elems = _R * _N_CLASSES

    x_flat = x.reshape(1, rows)

    @pl.kernel(
        out_type=jax.ShapeDtypeStruct((rows * _N_CLASSES,), jnp.float32),
        mesh=plsc.VectorSubcoreMesh(core_axis_name="c", subcore_axis_name="s"),
        compiler_params=pltpu.CompilerParams(needs_layout_passes=False),
        scratch_types=[
            pltpu.VMEM((chunk_elems,), jnp.float32),
            pltpu.VMEM((chunk_elems,), jnp.float32),
            pltpu.VMEM((1, rows_per_worker), jnp.int32),
            pltpu.SemaphoreType.DMA((2,)),
            pltpu.SemaphoreType.DMA,
        ],
    )
    def sc_onehot(x_hbm, o_hbm, zb0, zb1, xv, zsem, xsem):
        core = jax.lax.axis_index("c")
        sub = jax.lax.axis_index("s")
        wid = core * 16 + sub
        row0 = wid * rows_per_worker
        elem0 = row0 * _N_CLASSES
        iota = jax.lax.broadcasted_iota(jnp.int32, (_LANES,), 0)
        ones = jnp.ones((_LANES,), jnp.float32)
        zeros = jnp.zeros((_LANES,), jnp.float32)

        pltpu.make_async_copy(
            x_hbm.at[0, pl.ds(row0, rows_per_worker)],
            xv.at[0],
            xsem,
        ).start()

        def zfill(zb):
            @pl.loop(0, chunk_elems, step=_LANES)
            def _z(i):
                zb[pl.ds(i, _LANES)] = zeros

        zfill(zb0)
        zfill(zb1)

        def zcopy(c, zb, slot):
            return pltpu.make_async_copy(
                zb,
                o_hbm.at[pl.ds(elem0 + c * chunk_elems, chunk_elems)],
                zsem.at[slot],
            )

        def patch(zb, c, values):
            @pl.loop(0, _R, step=_LANES)
            def _g(g):
                cvec = xv[0, pl.ds(c * _R + g, _LANES)]
                flat = (iota + g) * _N_CLASSES + cvec
                plsc.store_scatter(zb, [flat], values)

        pltpu.make_async_copy(
            x_hbm.at[0, pl.ds(row0, rows_per_worker)], xv.at[0], xsem
        ).wait()

        def chunk_body(zb, slot, c):
            @pl.when(c >= 2)
            def _reset():
                zcopy(c - 2, zb, slot).wait()
                patch(zb, c - 2, zeros)

            patch(zb, c, ones)
            zcopy(c, zb, slot).start()

        @pl.loop(0, n_chunks)
        def _c(c):
            slot = jax.lax.rem(c, 2)

            @pl.when(slot == 0)
            def _s0():
                chunk_body(zb0, 0, c)

            @pl.when(slot == 1)
            def _s1():
                chunk_body(zb1, 1, c)

        zcopy(n_chunks - 2, zb0, 0).wait()
        zcopy(n_chunks - 1, zb1, 1).wait()

    return sc_onehot(x_flat).reshape(b, s, _N_CLASSES)
